# baseline (speedup 1.0000x reference)
import functools

import jax
import jax.numpy as jnp
from jax import lax
from jax.experimental import pallas as pl
from jax.experimental.pallas import tpu as pltpu
from jax.experimental.pallas import tpu_sc as plsc

_ROWS = 4096 * 200
_NW = 32
_RPW = _ROWS // _NW
_CHUNK = _RPW // 2


def _body(x_hbm, out_hbm, buf):
    wid = lax.axis_index("s") * 2 + lax.axis_index("c")
    for i in range(_RPW // _CHUNK):
        base = wid * _RPW + i * _CHUNK
        pltpu.sync_copy(x_hbm.at[pl.ds(base, _CHUNK), pl.ds(0, 2)], buf)
        pltpu.sync_copy(buf, out_hbm.at[pl.ds(base, _CHUNK)])


def kernel(x):
    b, s, c = x.shape
    x2 = x.reshape(b * s, c)
    mesh = plsc.VectorSubcoreMesh(core_axis_name="c", subcore_axis_name="s")
    run = functools.partial(
        pl.kernel,
        out_type=jax.ShapeDtypeStruct((b * s, 2), jnp.float32),
        mesh=mesh,
        scratch_types=[pltpu.VMEM((_CHUNK, 2), jnp.float32)],
        compiler_params=pltpu.CompilerParams(use_tc_tiling_on_sc=False),
    )(_body)
    out = run(x2)
    return out.reshape(b, s, 2)
